# baseline (speedup 1.0000x reference)
import functools
import math

import jax
import jax.numpy as jnp
from jax import lax
from jax.experimental import pallas as pl
from jax.experimental.pallas import tpu as pltpu
from jax.experimental.pallas import tpu_sc as plsc

B, S, D, E = 4096, 8, 2048, 64
_NUM_DIST_BUCKETS = 3
_INV = 1.0 / math.sqrt(D)


_GATHER_WORKERS = 8
_ROWS_PER_W = E // _GATHER_WORKERS


def _sc_body(emb_hbm, cent_hbm, nidx_hbm, out_emb, out_cent,
             idx_v, rows_v, sem_a, sem_b):
    wid = lax.axis_index("s") * 2 + lax.axis_index("c")

    @pl.when(wid < _GATHER_WORKERS)
    def _gather():
        base = wid * _ROWS_PER_W
        pltpu.sync_copy(nidx_hbm.at[pl.ds(base, _ROWS_PER_W)], idx_v)
        pltpu.async_copy(emb_hbm.at[idx_v], rows_v, sem_a).wait()
        pltpu.sync_copy(rows_v, out_emb.at[pl.ds(base, _ROWS_PER_W)])
        pltpu.async_copy(cent_hbm.at[idx_v], rows_v, sem_b).wait()
        pltpu.sync_copy(rows_v, out_cent.at[pl.ds(base, _ROWS_PER_W)])


def _sc_prep(expert_embeddings, centrality_embeddings, next_indices):
    mesh = plsc.VectorSubcoreMesh(core_axis_name="c", subcore_axis_name="s")
    fn = functools.partial(
        pl.kernel,
        mesh=mesh,
        out_type=[
            jax.ShapeDtypeStruct((E, D), jnp.float32),
            jax.ShapeDtypeStruct((E, D), jnp.float32),
        ],
        scratch_types=[
            pltpu.VMEM((_ROWS_PER_W,), jnp.int32),
            pltpu.VMEM((_ROWS_PER_W, D), jnp.float32),
            pltpu.SemaphoreType.DMA,
            pltpu.SemaphoreType.DMA,
        ],
    )(_sc_body)
    return fn(expert_embeddings, centrality_embeddings, next_indices)


_BBLK = 512
_NXBUF = 3
_WCHUNK = 512
_NWCHUNK = D // _WCHUNK


def _main_body(x_hbm, w_hbm, emb_ref, cent_ref, lns_ref, lnb_ref,
               out_ref, lnbq_ref,
               xbuf, wbuf, q_ref, xsem, wsem):
    i = pl.program_id(0)
    nb = pl.num_programs(0)

    def _xcopy(idx, slot):
        return pltpu.make_async_copy(
            x_hbm.at[pl.ds(idx * _BBLK, _BBLK), S - 1, :],
            xbuf.at[slot], xsem.at[slot])

    @pl.when(i == 0)
    def _prep():
        for s in range(_NXBUF):
            _xcopy(s, s).start()
        for c in range(_NWCHUNK):
            pltpu.make_async_copy(w_hbm.at[pl.ds(c * _WCHUNK, _WCHUNK), :],
                                  wbuf.at[c], wsem.at[c]).start()
        a = emb_ref[...] + cent_ref[...]
        for c in range(_NWCHUNK):
            pltpu.make_async_copy(w_hbm.at[pl.ds(c * _WCHUNK, _WCHUNK), :],
                                  wbuf.at[c], wsem.at[c]).wait()
            a_c = a[:, c * _WCHUNK:(c + 1) * _WCHUNK]
            part = lax.dot_general(wbuf[c], a_c, (((0,), (1,)), ((), ())),
                                   preferred_element_type=jnp.float32)
            if c == 0:
                q_ref[...] = part
            else:
                q_ref[...] += part
        lnbq_ref[...] = lax.dot_general(lnb_ref[...], q_ref[...],
                                        (((1,), (0,)), ((), ())),
                                        preferred_element_type=jnp.float32)
        q_ref[...] = q_ref[...] * lns_ref[...] * _INV

    @pl.when(jnp.logical_and(i >= 1, i + _NXBUF - 1 < nb))
    def _prefetch():
        idx = i + _NXBUF - 1
        _xcopy(idx, idx % _NXBUF).start()

    _xcopy(i, i % _NXBUF).wait()
    x = xbuf[i % _NXBUF]
    mu = jnp.mean(x, axis=1, keepdims=True)
    xc = x - mu
    var = jnp.mean(xc * xc, axis=1, keepdims=True)
    z = xc * lax.rsqrt(var + 1e-5)
    out_ref[...] = jnp.dot(z, q_ref[...],
                           preferred_element_type=jnp.float32)


def _tc_main(hidden_states, W, expert_embeddings, centrality_embeddings,
             ln_scale, ln_bias):
    grid = (B // _BBLK,)
    return pl.pallas_call(
        _main_body,
        grid=grid,
        in_specs=[
            pl.BlockSpec(memory_space=pltpu.MemorySpace.HBM),
            pl.BlockSpec(memory_space=pltpu.MemorySpace.HBM),
            pl.BlockSpec((E, D), lambda i: (0, 0)),
            pl.BlockSpec((E, D), lambda i: (0, 0)),
            pl.BlockSpec((D, 1), lambda i: (0, 0)),
            pl.BlockSpec((1, D), lambda i: (0, 0)),
        ],
        out_specs=[
            pl.BlockSpec((_BBLK, E), lambda i: (i, 0)),
            pl.BlockSpec((1, E), lambda i: (0, 0)),
        ],
        out_shape=[
            jax.ShapeDtypeStruct((B, E), jnp.float32),
            jax.ShapeDtypeStruct((1, E), jnp.float32),
        ],
        scratch_shapes=[
            pltpu.VMEM((_NXBUF, _BBLK, D), jnp.float32),
            pltpu.VMEM((_NWCHUNK, _WCHUNK, D), jnp.float32),
            pltpu.VMEM((D, E), jnp.float32),
            pltpu.SemaphoreType.DMA((_NXBUF,)),
            pltpu.SemaphoreType.DMA((_NWCHUNK,)),
        ],
    )(hidden_states, W, expert_embeddings, centrality_embeddings,
      ln_scale.reshape(D, 1), ln_bias.reshape(1, D))


def _epilogue_body(raw_ref, lnbq_ref, emb_ref, cent_ref, b_ref,
                   nidx_ref, distt_ref, spat_ref, cur_ref, out_ref):
    iota0 = lax.broadcasted_iota(jnp.int32, (E, E), 0)
    ohm = iota0 == nidx_ref[...]
    ohc = jnp.where(ohm, 1.0, 0.0)
    en = emb_ref[...] + cent_ref[...]
    ben = lax.dot_general(b_ref[...], en, (((1,), (1,)), ((), ())),
                          preferred_element_type=jnp.float32)
    lnbq_g = jnp.dot(lnbq_ref[...], ohc,
                     preferred_element_type=jnp.float32)
    cur_s = cur_ref[0, 0]
    iota1 = lax.broadcasted_iota(jnp.int32, (E, E), 1)
    rowc = jnp.sum(jnp.where(iota1 == cur_s, distt_ref[...], 0),
                   axis=1, keepdims=True)
    d_row = jnp.sum(jnp.where(ohm, rowc, 0), axis=0, keepdims=True)
    sb = jnp.zeros((1, E), jnp.float32)
    for k in range(_NUM_DIST_BUCKETS):
        sb = sb + jnp.where(d_row == k, spat_ref[0, k], 0.0)
    bias = (ben + lnbq_g + sb) * _INV
    out_ref[...] = jnp.dot(raw_ref[...], ohc,
                           preferred_element_type=jnp.float32) + bias


def _tc_epilogue(raw, lnbq, emb_g, cent_g, b, next_indices,
                 expert_distances, spatial_embeddings, current_expert_idx):
    return pl.pallas_call(
        _epilogue_body,
        grid=(1,),
        in_specs=[
            pl.BlockSpec((B, E), lambda i: (0, 0)),
            pl.BlockSpec((1, E), lambda i: (0, 0)),
            pl.BlockSpec((E, D), lambda i: (0, 0)),
            pl.BlockSpec((E, D), lambda i: (0, 0)),
            pl.BlockSpec((1, D), lambda i: (0, 0)),
            pl.BlockSpec((1, E), lambda i: (0, 0)),
            pl.BlockSpec((E, E), lambda i: (0, 0)),
            pl.BlockSpec((1, _NUM_DIST_BUCKETS), lambda i: (0, 0)),
            pl.BlockSpec((1, 1), lambda i: (0, 0)),
        ],
        out_specs=pl.BlockSpec((B, E), lambda i: (0, 0)),
        out_shape=jax.ShapeDtypeStruct((B, E), jnp.float32),
    )(raw, lnbq, emb_g, cent_g, b.reshape(1, D),
      next_indices.reshape(1, E).astype(jnp.int32),
      expert_distances.T.astype(jnp.int32),
      spatial_embeddings.reshape(1, _NUM_DIST_BUCKETS).astype(jnp.float32),
      jnp.asarray(current_expert_idx, jnp.int32).reshape(1, 1))


def kernel(hidden_states, expert_embeddings, centrality_embeddings,
           spatial_embeddings, ln_scale, ln_bias, W, b, next_indices,
           expert_distances, current_expert_idx):
    emb_g, cent_g = _sc_prep(
        expert_embeddings, centrality_embeddings, next_indices)
    raw, lnbq = _tc_main(hidden_states, W, expert_embeddings,
                         centrality_embeddings, ln_scale, ln_bias)
    return _tc_epilogue(raw, lnbq, emb_g, cent_g, b, next_indices,
                        expert_distances, spatial_embeddings,
                        current_expert_idx)


# speedup vs baseline: 1.0537x; 1.0537x over previous
import functools
import math

import jax
import jax.numpy as jnp
from jax import lax
from jax.experimental import pallas as pl
from jax.experimental.pallas import tpu as pltpu
from jax.experimental.pallas import tpu_sc as plsc

B, S, D, E = 4096, 8, 2048, 64
_NUM_DIST_BUCKETS = 3
_INV = 1.0 / math.sqrt(D)


_GATHER_WORKERS = 8
_ROWS_PER_W = E // _GATHER_WORKERS


def _sc_body(emb_hbm, cent_hbm, nidx_hbm, out_emb, out_cent,
             idx_v, rows_v, sem_a, sem_b):
    wid = lax.axis_index("s") * 2 + lax.axis_index("c")

    @pl.when(wid < _GATHER_WORKERS)
    def _gather():
        base = wid * _ROWS_PER_W
        pltpu.sync_copy(nidx_hbm.at[pl.ds(base, _ROWS_PER_W)], idx_v)
        pltpu.async_copy(emb_hbm.at[idx_v], rows_v, sem_a).wait()
        pltpu.sync_copy(rows_v, out_emb.at[pl.ds(base, _ROWS_PER_W)])
        pltpu.async_copy(cent_hbm.at[idx_v], rows_v, sem_b).wait()
        pltpu.sync_copy(rows_v, out_cent.at[pl.ds(base, _ROWS_PER_W)])


def _sc_prep(expert_embeddings, centrality_embeddings, next_indices):
    mesh = plsc.VectorSubcoreMesh(core_axis_name="c", subcore_axis_name="s")
    fn = functools.partial(
        pl.kernel,
        mesh=mesh,
        out_type=[
            jax.ShapeDtypeStruct((E, D), jnp.float32),
            jax.ShapeDtypeStruct((E, D), jnp.float32),
        ],
        scratch_types=[
            pltpu.VMEM((_ROWS_PER_W,), jnp.int32),
            pltpu.VMEM((_ROWS_PER_W, D), jnp.float32),
            pltpu.SemaphoreType.DMA,
            pltpu.SemaphoreType.DMA,
        ],
    )(_sc_body)
    return fn(expert_embeddings, centrality_embeddings, next_indices)


_BBLK = 512
_NXBUF = 3
_WCHUNK = 512
_NWCHUNK = D // _WCHUNK


def _main_body(x_hbm, w_hbm, emb_ref, cent_ref, lns_ref, lnb_ref,
               nidx_ref, dist_ref, spat_ref, cur_ref,
               out_ref, brest_ref,
               xbuf, wbuf, q_ref, csum_ref, xsem, wsem):
    i = pl.program_id(0)
    nb = pl.num_programs(0)

    def _xcopy(idx, slot):
        return pltpu.make_async_copy(
            x_hbm.at[pl.ds(idx * _BBLK, _BBLK), S - 1, :],
            xbuf.at[slot], xsem.at[slot])

    @pl.when(i == 0)
    def _prep():
        for s in range(_NXBUF):
            _xcopy(s, s).start()
        for c in range(_NWCHUNK):
            pltpu.make_async_copy(w_hbm.at[pl.ds(c * _WCHUNK, _WCHUNK), :],
                                  wbuf.at[c], wsem.at[c]).start()
        a = emb_ref[...] + cent_ref[...]
        for c in range(_NWCHUNK):
            pltpu.make_async_copy(w_hbm.at[pl.ds(c * _WCHUNK, _WCHUNK), :],
                                  wbuf.at[c], wsem.at[c]).wait()
            a_c = a[:, c * _WCHUNK:(c + 1) * _WCHUNK]
            part = lax.dot_general(wbuf[c], a_c, (((0,), (1,)), ((), ())),
                                   preferred_element_type=jnp.float32)
            if c == 0:
                q_ref[...] = part
            else:
                q_ref[...] += part
        iota0 = lax.broadcasted_iota(jnp.int32, (E, E), 0)
        ohc = jnp.where(iota0 == nidx_ref[...], 1.0, 0.0)
        q_ref[...] = jnp.dot(q_ref[...], ohc,
                             preferred_element_type=jnp.float32)
        lnbq = lax.dot_general(lnb_ref[...], q_ref[...],
                               (((1,), (0,)), ((), ())),
                               preferred_element_type=jnp.float32)
        cur_s = cur_ref[0, 0]
        rowdist = jnp.sum(
            jnp.where(iota0 == cur_s, dist_ref[...].astype(jnp.float32), 0.0),
            axis=0, keepdims=True)
        d_row = jnp.dot(rowdist, ohc,
                        preferred_element_type=jnp.float32)
        sb = jnp.zeros((1, E), jnp.float32)
        for k in range(_NUM_DIST_BUCKETS):
            sb = sb + jnp.where(d_row == float(k), spat_ref[0, k], 0.0)
        brest_ref[...] = (lnbq + sb) * _INV
        q_ref[...] = q_ref[...] * lns_ref[...] * _INV
        csum_ref[...] = jnp.sum(q_ref[...], axis=0, keepdims=True)

    @pl.when(jnp.logical_and(i >= 1, i + _NXBUF - 1 < nb))
    def _prefetch():
        idx = i + _NXBUF - 1
        _xcopy(idx, idx % _NXBUF).start()

    _xcopy(i, i % _NXBUF).wait()
    x = xbuf[i % _NXBUF]
    s1 = jnp.mean(x, axis=1, keepdims=True)
    s2 = jnp.mean(x * x, axis=1, keepdims=True)
    r = lax.rsqrt(s2 - s1 * s1 + 1e-5)
    y = jnp.dot(x, q_ref[...], preferred_element_type=jnp.float32)
    out_ref[...] = y * r - (s1 * r) * csum_ref[...]


def _tc_main(hidden_states, W, expert_embeddings, centrality_embeddings,
             ln_scale, ln_bias, next_indices, expert_distances,
             spatial_embeddings, current_expert_idx):
    grid = (B // _BBLK,)
    return pl.pallas_call(
        _main_body,
        grid=grid,
        in_specs=[
            pl.BlockSpec(memory_space=pltpu.MemorySpace.HBM),
            pl.BlockSpec(memory_space=pltpu.MemorySpace.HBM),
            pl.BlockSpec((E, D), lambda i: (0, 0)),
            pl.BlockSpec((E, D), lambda i: (0, 0)),
            pl.BlockSpec((D, 1), lambda i: (0, 0)),
            pl.BlockSpec((1, D), lambda i: (0, 0)),
            pl.BlockSpec((1, E), lambda i: (0, 0)),
            pl.BlockSpec((E, E), lambda i: (0, 0)),
            pl.BlockSpec((1, _NUM_DIST_BUCKETS), lambda i: (0, 0)),
            pl.BlockSpec((1, 1), lambda i: (0, 0)),
        ],
        out_specs=[
            pl.BlockSpec((_BBLK, E), lambda i: (i, 0)),
            pl.BlockSpec((1, E), lambda i: (0, 0)),
        ],
        out_shape=[
            jax.ShapeDtypeStruct((B, E), jnp.float32),
            jax.ShapeDtypeStruct((1, E), jnp.float32),
        ],
        scratch_shapes=[
            pltpu.VMEM((_NXBUF, _BBLK, D), jnp.float32),
            pltpu.VMEM((_NWCHUNK, _WCHUNK, D), jnp.float32),
            pltpu.VMEM((D, E), jnp.float32),
            pltpu.VMEM((1, E), jnp.float32),
            pltpu.SemaphoreType.DMA((_NXBUF,)),
            pltpu.SemaphoreType.DMA((_NWCHUNK,)),
        ],
    )(hidden_states, W, expert_embeddings, centrality_embeddings,
      ln_scale.reshape(D, 1), ln_bias.reshape(1, D),
      next_indices.reshape(1, E).astype(jnp.int32),
      expert_distances.astype(jnp.int32),
      spatial_embeddings.reshape(1, _NUM_DIST_BUCKETS).astype(jnp.float32),
      jnp.asarray(current_expert_idx, jnp.int32).reshape(1, 1))


def _epilogue_body(raw_ref, brest_ref, emb_ref, cent_ref, b_ref, out_ref):
    en = emb_ref[...] + cent_ref[...]
    ben = lax.dot_general(b_ref[...], en, (((1,), (1,)), ((), ())),
                          preferred_element_type=jnp.float32)
    out_ref[...] = raw_ref[...] + (brest_ref[...] + ben * _INV)


def _tc_epilogue(raw, brest, emb_g, cent_g, b):
    return pl.pallas_call(
        _epilogue_body,
        grid=(1,),
        in_specs=[
            pl.BlockSpec((B, E), lambda i: (0, 0)),
            pl.BlockSpec((1, E), lambda i: (0, 0)),
            pl.BlockSpec((E, D), lambda i: (0, 0)),
            pl.BlockSpec((E, D), lambda i: (0, 0)),
            pl.BlockSpec((1, D), lambda i: (0, 0)),
        ],
        out_specs=pl.BlockSpec((B, E), lambda i: (0, 0)),
        out_shape=jax.ShapeDtypeStruct((B, E), jnp.float32),
    )(raw, brest, emb_g, cent_g, b.reshape(1, D))


def kernel(hidden_states, expert_embeddings, centrality_embeddings,
           spatial_embeddings, ln_scale, ln_bias, W, b, next_indices,
           expert_distances, current_expert_idx):
    emb_g, cent_g = _sc_prep(
        expert_embeddings, centrality_embeddings, next_indices)
    raw, brest = _tc_main(hidden_states, W, expert_embeddings,
                          centrality_embeddings, ln_scale, ln_bias,
                          next_indices, expert_distances,
                          spatial_embeddings, current_expert_idx)
    return _tc_epilogue(raw, brest, emb_g, cent_g, b)
